# baseline (speedup 1.0000x reference)
import functools

import jax
import jax.numpy as jnp
from jax import lax
from jax.experimental import pallas as pl
from jax.experimental.pallas import tpu as pltpu
from jax.experimental.pallas import tpu_sc as plsc

_D = 64
_SEQ = 200
_B = 4096
_NCLS = 10
_NC = 2
_NS = 16
_NW = _NC * _NS
_BPW = _B // _NW
_CH = 2
_CHL = _SEQ // _CH
_LANES = 16

_mesh = plsc.VectorSubcoreMesh(core_axis_name="c", subcore_axis_name="s")


@functools.partial(
    pl.kernel,
    out_type=jax.ShapeDtypeStruct((_B, _D), jnp.float32),
    mesh=_mesh,
    scratch_types=[
        pltpu.VMEM((_BPW, _CH, _CHL), jnp.int32),
        pltpu.VMEM((2, _SEQ, _D), jnp.float32),
        pltpu.VMEM((_BPW, _D), jnp.float32),
        pltpu.SemaphoreType.DMA,
        pltpu.SemaphoreType.DMA,
    ],
    compiler_params=pltpu.CompilerParams(use_tc_tiling_on_sc=False),
)
def _sc_pool(xt_hbm, table_hbm, out_hbm, idx_v, rows_v, pooled_v, sem0, sem1):
    wid = lax.axis_index("s") * _NC + lax.axis_index("c")
    base = wid * _BPW
    pltpu.sync_copy(xt_hbm.at[pl.ds(base, _BPW)], idx_v)

    sems = (sem0, sem1)

    def start(e, slot):
        for c in range(_CH):
            pltpu.async_copy(
                table_hbm.at[idx_v.at[e, c]],
                rows_v.at[slot, pl.ds(c * _CHL, _CHL)],
                sems[slot],
            )

    def drain(slot):
        pltpu.make_async_copy(
            table_hbm.at[pl.ds(0, _SEQ)], rows_v.at[slot], sems[slot]
        ).wait()

    def accum(e, slot):
        def body(r, acc):
            return tuple(
                acc[i] + rows_v[slot, r, pl.ds(i * _LANES, _LANES)]
                for i in range(_D // _LANES)
            )

        z = jnp.zeros((_LANES,), jnp.float32)
        acc = lax.fori_loop(0, _SEQ, body, (z,) * (_D // _LANES))
        for i in range(_D // _LANES):
            pooled_v[e, pl.ds(i * _LANES, _LANES)] = acc[i] * (1.0 / _SEQ)

    start(0, 0)

    def outer(g, carry):
        for slot in range(2):
            e = g * 2 + slot
            drain(slot)

            @pl.when(e + 1 < _BPW)
            def _():
                start(e + 1, (slot + 1) % 2)

            accum(e, slot)
        return carry

    lax.fori_loop(0, _BPW // 2, outer, 0)
    pltpu.sync_copy(pooled_v, out_hbm.at[pl.ds(base, _BPW)])


def _tc_head(p_ref, wt_ref, b_ref, o_ref):
    o_ref[...] = (
        jnp.dot(p_ref[...], wt_ref[...], preferred_element_type=jnp.float32)
        + b_ref[...]
    )


def kernel(x, table, W, b):
    xt = x.T.reshape(_B, _CH, _CHL).astype(jnp.int32)
    pooled = _sc_pool(xt, table)
    return pl.pallas_call(
        _tc_head,
        out_shape=jax.ShapeDtypeStruct((_B, _NCLS), jnp.float32),
    )(pooled, W.T, b.reshape(1, _NCLS))


# speedup vs baseline: 1.0459x; 1.0459x over previous
import functools

import jax
import jax.numpy as jnp
from jax import lax
from jax.experimental import pallas as pl
from jax.experimental.pallas import tpu as pltpu
from jax.experimental.pallas import tpu_sc as plsc

_D = 64
_SEQ = 200
_B = 4096
_NCLS = 10
_NC = 2
_NS = 16
_NW = _NC * _NS
_BPW = _B // _NW
_LANES = 16
_NSLOT = 4

_mesh = plsc.VectorSubcoreMesh(core_axis_name="c", subcore_axis_name="s")


@functools.partial(
    pl.kernel,
    out_type=jax.ShapeDtypeStruct((_B, _D), jnp.float32),
    mesh=_mesh,
    scratch_types=[
        pltpu.VMEM((_SEQ, _BPW), jnp.int32),
        pltpu.VMEM((_NSLOT, _BPW, _D), jnp.float32),
        pltpu.VMEM((_BPW, _D), jnp.float32),
        pltpu.SemaphoreType.DMA,
        pltpu.SemaphoreType.DMA,
        pltpu.SemaphoreType.DMA,
        pltpu.SemaphoreType.DMA,
    ],
    compiler_params=pltpu.CompilerParams(use_tc_tiling_on_sc=False),
)
def _sc_pool(x_hbm, table_hbm, out_hbm, idx_v, rows_v, pooled_v, s0, s1, s2, s3):
    wid = lax.axis_index("s") * _NC + lax.axis_index("c")
    base = wid * _BPW
    pltpu.sync_copy(x_hbm.at[:, pl.ds(base, _BPW)], idx_v)

    sems = (s0, s1, s2, s3)

    def start(r, slot):
        pltpu.async_copy(
            table_hbm.at[idx_v.at[r]], rows_v.at[slot], sems[slot]
        )

    def drain(slot):
        pltpu.make_async_copy(
            table_hbm.at[pl.ds(0, _BPW)], rows_v.at[slot], sems[slot]
        ).wait()

    def zero_body(j, carry):
        for i in range(_D // _LANES):
            pooled_v[j, pl.ds(i * _LANES, _LANES)] = jnp.zeros(
                (_LANES,), jnp.float32
            )
        return carry

    lax.fori_loop(0, _BPW, zero_body, 0)

    for slot in range(_NSLOT - 1):
        start(slot, slot)

    def accum(slot):
        def body(j, carry):
            for i in range(_D // _LANES):
                plsc.addupdate(
                    pooled_v.at[j, pl.ds(i * _LANES, _LANES)],
                    rows_v[slot, j, pl.ds(i * _LANES, _LANES)],
                )
            return carry

        lax.fori_loop(0, _BPW, body, 0)

    def outer(g, carry):
        for k in range(_NSLOT):
            r = g * _NSLOT + k
            drain(k)

            @pl.when(r + _NSLOT - 1 < _SEQ)
            def _():
                start(r + _NSLOT - 1, (k + _NSLOT - 1) % _NSLOT)

            accum(k)
        return carry

    lax.fori_loop(0, _SEQ // _NSLOT, outer, 0)

    pltpu.sync_copy(pooled_v, out_hbm.at[pl.ds(base, _BPW)])


def _tc_head(p_ref, wt_ref, b_ref, o_ref):
    o_ref[...] = (
        jnp.dot(p_ref[...], wt_ref[...], preferred_element_type=jnp.float32)
        * (1.0 / _SEQ)
        + b_ref[...]
    )


def kernel(x, table, W, b):
    pooled = _sc_pool(x.astype(jnp.int32), table)
    return pl.pallas_call(
        _tc_head,
        out_shape=jax.ShapeDtypeStruct((_B, _NCLS), jnp.float32),
    )(pooled, W.T, b.reshape(1, _NCLS))


# speedup vs baseline: 1.3142x; 1.2564x over previous
import functools

import jax
import jax.numpy as jnp
from jax import lax
from jax.experimental import pallas as pl
from jax.experimental.pallas import tpu as pltpu
from jax.experimental.pallas import tpu_sc as plsc

_V = 1000000
_D = 64
_SEQ = 200
_B = 4096
_NCLS = 10
_NC = 2
_NS = 16
_NW = _NC * _NS
_BPW = _B // _NW
_LANES = 16
_NSLOT = 4

_CBLK = 2048
_HBLK = _CBLK // 2
_NBLK = (_V + _CBLK - 1) // _CBLK
_LINR = _NBLK * _HBLK
_DECL = 2 * _LINR

_mesh = plsc.VectorSubcoreMesh(core_axis_name="c", subcore_axis_name="s")


def _tc_repack(t_ref, o_ref):
    tt = t_ref[...].T
    o_ref[:, 0:_D] = tt[0:_HBLK]
    o_ref[:, _D : 2 * _D] = tt[_HBLK:_CBLK]


@functools.partial(
    pl.kernel,
    out_type=jax.ShapeDtypeStruct((_B, _D), jnp.float32),
    mesh=_mesh,
    scratch_types=[
        pltpu.VMEM((_SEQ, _BPW), jnp.int32),
        pltpu.VMEM((_NSLOT, _BPW, _D), jnp.float32),
        pltpu.VMEM((_BPW, _D), jnp.float32),
        pltpu.SemaphoreType.DMA,
        pltpu.SemaphoreType.DMA,
        pltpu.SemaphoreType.DMA,
        pltpu.SemaphoreType.DMA,
    ],
    compiler_params=pltpu.CompilerParams(use_tc_tiling_on_sc=False),
)
def _sc_pool(x_hbm, table_hbm, out_hbm, idx_v, rows_v, pooled_v, s0, s1, s2, s3):
    wid = lax.axis_index("s") * _NC + lax.axis_index("c")
    base = wid * _BPW
    pltpu.sync_copy(x_hbm.at[:, pl.ds(base, _BPW)], idx_v)

    def xform_body(s, carry):
        for l in range(_BPW // _LANES):
            v = idx_v[s, pl.ds(l * _LANES, _LANES)]
            k = v & (_CBLK - 1)
            idx_v[s, pl.ds(l * _LANES, _LANES)] = (
                (v - k) + ((k & (_HBLK - 1)) << 1) + (k >> 10)
            )
        return carry

    lax.fori_loop(0, _SEQ, xform_body, 0)

    sems = (s0, s1, s2, s3)

    def start(r, slot):
        pltpu.async_copy(
            table_hbm.at[idx_v.at[r]], rows_v.at[slot], sems[slot]
        )

    def drain(slot):
        pltpu.make_async_copy(
            table_hbm.at[pl.ds(0, _BPW)], rows_v.at[slot], sems[slot]
        ).wait()

    def zero_body(j, carry):
        for i in range(_D // _LANES):
            pooled_v[j, pl.ds(i * _LANES, _LANES)] = jnp.zeros(
                (_LANES,), jnp.float32
            )
        return carry

    lax.fori_loop(0, _BPW, zero_body, 0)

    for slot in range(_NSLOT - 1):
        start(slot, slot)

    def accum(slot):
        def body(j, carry):
            for i in range(_D // _LANES):
                plsc.addupdate(
                    pooled_v.at[j, pl.ds(i * _LANES, _LANES)],
                    rows_v[slot, j, pl.ds(i * _LANES, _LANES)],
                )
            return carry

        lax.fori_loop(0, _BPW, body, 0)

    def outer(g, carry):
        for k in range(_NSLOT):
            r = g * _NSLOT + k
            drain(k)

            @pl.when(r + _NSLOT - 1 < _SEQ)
            def _():
                start(r + _NSLOT - 1, (k + _NSLOT - 1) % _NSLOT)

            accum(k)
        return carry

    lax.fori_loop(0, _SEQ // _NSLOT, outer, 0)

    pltpu.sync_copy(pooled_v, out_hbm.at[pl.ds(base, _BPW)])


def _tc_head(p_ref, wt_ref, b_ref, o_ref):
    o_ref[...] = (
        jnp.dot(p_ref[...], wt_ref[...], preferred_element_type=jnp.float32)
        * (1.0 / _SEQ)
        + b_ref[...]
    )


def kernel(x, table, W, b):
    lin = pl.pallas_call(
        _tc_repack,
        grid=(_NBLK,),
        in_specs=[pl.BlockSpec((_D, _CBLK), lambda i: (0, i))],
        out_specs=pl.BlockSpec((_HBLK, 2 * _D), lambda i: (i, 0)),
        out_shape=jax.ShapeDtypeStruct((_LINR, 2 * _D), jnp.float32),
    )(table.T)
    tab_lin = lin.reshape(_DECL, _D)
    pooled = _sc_pool(x.astype(jnp.int32), tab_lin)
    return pl.pallas_call(
        _tc_head,
        out_shape=jax.ShapeDtypeStruct((_B, _NCLS), jnp.float32),
    )(pooled, W.T, b.reshape(1, _NCLS))
